# baseline (speedup 1.0000x reference)
import jax
import jax.numpy as jnp
from jax import lax
from jax.experimental import pallas as pl
from jax.experimental.pallas import tpu as pltpu
from jax.experimental.pallas import tpu_sc as plsc

B = 16384
D = 64
NC = 2
NS = 16
L = 16
NW = NC * NS
BPW = B // NW

NU = 1000000
NI = 100000
RB = 8
UCW = 512
UTAIL = (NU // 128) * 128
NUCH = UTAIL // UCW
KU = -(-NUCH // NW)
LCAP = 1024
ACAP = 1024
BIG = 1 << 30


def _main_body(users_hbm, items_hbm, utab_hbm, itab_hbm, part_hbm,
               ubuild, ai, ulist, ubl, acol, ab, uchunk0, uchunk1,
               vblock, scores, cnts, semr0, semr1, semv):
    wid = lax.axis_index("s") * NC + lax.axis_index("c")
    iota16 = lax.iota(jnp.int32, L)

    pltpu.sync_copy(items_hbm, ai)

    def zero(t, carry):
        scores[pl.ds(t * L, L)] = jnp.zeros((L,), jnp.float32)
        return carry

    lax.fori_loop(0, (B + L) // L, zero, 0)

    cnts[0] = 0

    def build_outer(p, carry):
        pltpu.sync_copy(users_hbm.at[pl.ds(p * BPW, BPW)], ubuild)

        def build(t, carry2):
            b = p * BPW + t * L + iota16
            u = ubuild[pl.ds(t * L, L)]
            mu = (((u >> 9) & 31) == wid) & (u < UTAIL)
            cu = cnts[0]
            plsc.store_compressed(ulist.at[pl.ds(cu, L)], u, mask=mu)
            plsc.store_compressed(ubl.at[pl.ds(cu, L)], b, mask=mu)
            cnts[0] = cu + plsc.all_reduce_population_count(mu)[0]
            return carry2

        lax.fori_loop(0, BPW // L, build, 0)
        return carry

    lax.fori_loop(0, B // BPW, build_outer, 0)
    ulist[pl.ds(cnts[0], L)] = jnp.full((L,), BIG, jnp.int32)
    cnt = cnts[0]
    nvreg = (cnt + L - 1) >> 4

    def issue(k, chunkbuf, semr):
        cid = wid + NW * k

        @pl.when(cid < NUCH)
        def _():
            off = pl.multiple_of(cid * UCW, UCW)
            pltpu.async_copy(utab_hbm.at[:, pl.ds(off, UCW)], chunkbuf, semr)

    def process(k, chunkbuf, semr):
        cid = wid + NW * k

        @pl.when(cid < NUCH)
        def _():
            pltpu.make_async_copy(
                utab_hbm.at[:, pl.ds(0, UCW)], chunkbuf, semr).wait()

            def scan(t, c2):
                lv = ulist[pl.ds(t * L, L)]
                bv = ubl[pl.ds(t * L, L)]
                m = (lv >> 9) == cid
                plsc.store_compressed(acol.at[pl.ds(c2, L)],
                                      lv & (UCW - 1), mask=m)
                plsc.store_compressed(ab.at[pl.ds(c2, L)], bv, mask=m)
                return c2 + plsc.all_reduce_population_count(m)[0]

            c2 = lax.fori_loop(0, nvreg, scan, 0)
            ngroup = (c2 + L - 1) >> 4

            def group(g, carry2):
                col = acol[pl.ds(g * L, L)] & (UCW - 1)
                bs = ab[pl.ds(g * L, L)] & (B - 1)
                gm = iota16 < (c2 - g * L)
                iv = plsc.load_gather(ai, [bs])
                ivb = iv >> 1
                ivh = (iv & 1) << 6
                pltpu.async_copy(itab_hbm.at[ivb], vblock, semv).wait()
                acc = jnp.zeros((L,), jnp.float32)
                for d in range(D):
                    dd = jnp.full((L,), d, jnp.int32)
                    u = plsc.load_gather(chunkbuf, [dd, col])
                    v = plsc.load_gather(vblock, [iota16, ivh + dd])
                    acc = acc + u * v
                bs_dump = jnp.where(gm, bs, B)
                plsc.store_scatter(scores, [bs_dump], acc)
                return carry2

            lax.fori_loop(0, ngroup, group, 0)

    issue(0, uchunk0, semr0)

    def step(t, carry):
        issue(2 * t + 1, uchunk1, semr1)
        process(2 * t, uchunk0, semr0)

        @pl.when(t + 1 < KU // 2)
        def _():
            issue(2 * t + 2, uchunk0, semr0)

        process(2 * t + 1, uchunk1, semr1)
        return carry

    lax.fori_loop(0, KU // 2, step, 0)

    pltpu.sync_copy(scores.at[pl.ds(0, B)], part_hbm.at[pl.ds(wid * B, B)])


def _reduce_body(users_hbm, items_hbm, part_hbm, utail_hbm, itab_hbm,
                 out_hbm, uidx, iidx, pbuf, utb, vtb, accv, semp, semv):
    wid = lax.axis_index("s") * NC + lax.axis_index("c")
    base = wid * BPW
    iota16 = lax.iota(jnp.int32, L)

    pltpu.sync_copy(users_hbm.at[pl.ds(base, BPW)], uidx)
    pltpu.sync_copy(items_hbm.at[pl.ds(base, BPW)], iidx)
    pltpu.sync_copy(utail_hbm, utb)

    for w2 in range(NW):
        pltpu.async_copy(
            part_hbm.at[pl.ds(w2 * B + base, BPW)], pbuf.at[w2], semp)
    for w2 in range(NW):
        pltpu.make_async_copy(
            part_hbm.at[pl.ds(0, BPW)], pbuf.at[w2], semp).wait()

    def sum_group(g, carry):
        acc = jnp.zeros((L,), jnp.float32)
        for w2 in range(NW):
            acc = acc + pbuf[w2, pl.ds(g * L, L)]
        accv[pl.ds(g * L, L)] = acc
        return carry

    lax.fori_loop(0, BPW // L, sum_group, 0)

    def tail_group(g, carry):
        uvec = uidx[pl.ds(g * L, L)]
        mu = uvec >= UTAIL

        @pl.when(plsc.all_reduce_population_count(mu)[0] > 0)
        def _():
            ivec = iidx[pl.ds(g * L, L)]
            ivb = ivec >> 1
            ivh = (ivec & 1) << 6
            tuc = jnp.maximum(uvec - UTAIL, 0)
            pltpu.async_copy(itab_hbm.at[ivb], vtb, semv).wait()
            acc = jnp.zeros((L,), jnp.float32)
            for d in range(D):
                dd = jnp.full((L,), d, jnp.int32)
                ut = plsc.load_gather(utb, [tuc, dd], mask=mu)
                vt = plsc.load_gather(vtb, [iota16, ivh + dd], mask=mu)
                acc = acc + ut * vt
            old = accv[pl.ds(g * L, L)]
            accv[pl.ds(g * L, L)] = jnp.where(mu, acc, old)

        return carry

    lax.fori_loop(0, BPW // L, tail_group, 0)

    pltpu.sync_copy(accv, out_hbm.at[pl.ds(base, BPW)])


def kernel(users, items, users_table, items_table):
    ut_t = users_table.T
    utail = users_table[UTAIL:]
    users = users.astype(jnp.int32)
    items = items.astype(jnp.int32)
    mesh = plsc.VectorSubcoreMesh(core_axis_name="c", subcore_axis_name="s")

    main = pl.kernel(
        _main_body,
        out_type=jax.ShapeDtypeStruct((NW * B,), jnp.float32),
        mesh=mesh,
        compiler_params=pltpu.CompilerParams(needs_layout_passes=False),
        scratch_types=[
            pltpu.VMEM((BPW,), jnp.int32),
            pltpu.VMEM((B,), jnp.int32),
            pltpu.VMEM((LCAP + L,), jnp.int32),
            pltpu.VMEM((LCAP + L,), jnp.int32),
            pltpu.VMEM((ACAP + L,), jnp.int32),
            pltpu.VMEM((ACAP + L,), jnp.int32),
            pltpu.VMEM((D, UCW), jnp.float32),
            pltpu.VMEM((D, UCW), jnp.float32),
            pltpu.VMEM((L, 2 * D), jnp.float32),
            pltpu.VMEM((B + L,), jnp.float32),
            pltpu.SMEM((4,), jnp.int32),
            pltpu.SemaphoreType.DMA,
            pltpu.SemaphoreType.DMA,
            pltpu.SemaphoreType.DMA,
        ],
    )
    it2 = items_table.reshape(NI // 2, 2 * D)
    partials = main(users, items, ut_t, it2)

    reduce = pl.kernel(
        _reduce_body,
        out_type=jax.ShapeDtypeStruct((B,), jnp.float32),
        mesh=mesh,
        compiler_params=pltpu.CompilerParams(needs_layout_passes=False),
        scratch_types=[
            pltpu.VMEM((BPW,), jnp.int32),
            pltpu.VMEM((BPW,), jnp.int32),
            pltpu.VMEM((NW, BPW), jnp.float32),
            pltpu.VMEM((D, D), jnp.float32),
            pltpu.VMEM((L, 2 * D), jnp.float32),
            pltpu.VMEM((BPW,), jnp.float32),
            pltpu.SemaphoreType.DMA,
            pltpu.SemaphoreType.DMA,
        ],
    )
    return reduce(users, items, partials, utail, it2)
